# baseline (speedup 1.0000x reference)
import functools

import jax
import jax.numpy as jnp
from jax import lax
from jax.experimental import pallas as pl
from jax.experimental.pallas import tpu as pltpu
from jax.experimental.pallas import tpu_sc as plsc

VOCAB = 1000000
DIM = 64
BATCH = 16384
NEG = 5

NC = 2
NS = 16
NW = NC * NS
L = 16

HD = DIM // 2
BPW = BATCH // NW
CH = 128
NCH = BPW // CH
NGRP = CH // L
UNROLL = 4

_HIMASK = -65536


def _softplus(x):
    z = jnp.exp(-jnp.abs(x))
    w = z / (z + 2.0)
    w2 = w * w
    p = 1.0 + w2 * (1.0 / 3.0 + w2 * (1.0 / 5.0 + w2 * (1.0 / 7.0 + w2 * (1.0 / 9.0))))
    return jnp.maximum(x, 0.0) + 2.0 * w * p


def _halves(x):
    lo = plsc.bitcast(lax.shift_left(x, 16), jnp.float32)
    hi = plsc.bitcast(lax.bitwise_and(x, jnp.full((L,), _HIMASK, jnp.int32)),
                      jnp.float32)
    return lo, hi


def _body(pos_u_hbm, pos_v_hbm, neg_hbm, u_hbm, v_hbm, out_hbm,
          idx_u, idx_v, idx_n,
          ru0, rv0, rn0, ru1, rv1, rn1, loss_v,
          su0, sv0, sn0, su1, sv1, sn1):
    bufs = ((ru0, rv0, rn0), (ru1, rv1, rn1))
    sems = ((su0, sv0, sn0), (su1, sv1, sn1))

    c_id = lax.axis_index("c")
    s_id = lax.axis_index("s")
    wid = s_id * NC + c_id
    base = wid * BPW

    pltpu.sync_copy(pos_u_hbm.at[pl.ds(base, BPW)], idx_u)
    pltpu.sync_copy(pos_v_hbm.at[pl.ds(base, BPW)], idx_v)
    pltpu.sync_copy(neg_hbm.at[pl.ds(base * NEG, BPW * NEG)], idx_n)

    lane = lax.iota(jnp.int32, L)
    loss = jnp.zeros((L,), jnp.float32)

    def start_fetch(c, s):
        ru, rv, rn = bufs[s]
        semu, semv, semn = sems[s]
        cps = [
            pltpu.async_copy(u_hbm.at[idx_u.at[pl.ds(c * CH, CH)]], ru, semu),
            pltpu.async_copy(v_hbm.at[idx_v.at[pl.ds(c * CH, CH)]], rv, semv),
        ]
        for j in range(NEG):
            cps.append(pltpu.async_copy(
                v_hbm.at[idx_n.at[pl.ds(c * CH * NEG + j * CH, CH)]],
                rn.at[pl.ds(j * CH, CH)], semn))
        return cps

    pend = {0: start_fetch(0, 0)}

    for c in range(NCH):
        s = c % 2
        if c + 1 < NCH:
            pend[c + 1] = start_fetch(c + 1, 1 - s)
        for cp in pend.pop(c):
            cp.wait()
        ru, rv, rn = bufs[s]

        def group(g, loss):
            rb = g * L + lane
            rbn = [rb * NEG + j for j in range(NEG)]

            def dstep(t, accs):
                ap, a0, a1, a2, a3, a4 = accs
                for q in range(UNROLL):
                    p = t * UNROLL + q
                    pc = jnp.broadcast_to(p, (L,))
                    ulo, uhi = _halves(plsc.load_gather(ru, [rb, pc]))
                    vlo, vhi = _halves(plsc.load_gather(rv, [rb, pc]))
                    ap = ap + ulo * vlo + uhi * vhi
                    n0l, n0h = _halves(plsc.load_gather(rn, [rbn[0], pc]))
                    a0 = a0 + n0l * ulo + n0h * uhi
                    n1l, n1h = _halves(plsc.load_gather(rn, [rbn[1], pc]))
                    a1 = a1 + n1l * ulo + n1h * uhi
                    n2l, n2h = _halves(plsc.load_gather(rn, [rbn[2], pc]))
                    a2 = a2 + n2l * ulo + n2h * uhi
                    n3l, n3h = _halves(plsc.load_gather(rn, [rbn[3], pc]))
                    a3 = a3 + n3l * ulo + n3h * uhi
                    n4l, n4h = _halves(plsc.load_gather(rn, [rbn[4], pc]))
                    a4 = a4 + n4l * ulo + n4h * uhi
                return ap, a0, a1, a2, a3, a4

            z = jnp.zeros((L,), jnp.float32)
            ap, a0, a1, a2, a3, a4 = lax.fori_loop(
                0, HD // UNROLL, dstep, (z,) * 6)

            loss = loss + _softplus(-jnp.clip(ap, -10.0, 10.0))
            for t in (a0, a1, a2, a3, a4):
                loss = loss + _softplus(jnp.clip(t, -10.0, 10.0))
            return loss

        loss = lax.fori_loop(0, NGRP, group, loss)

    loss_v[...] = loss
    pltpu.sync_copy(loss_v, out_hbm.at[wid])


_mesh = plsc.VectorSubcoreMesh(core_axis_name="c", subcore_axis_name="s")

_sgns = functools.partial(
    pl.kernel,
    mesh=_mesh,
    compiler_params=pltpu.CompilerParams(
        needs_layout_passes=False, use_tc_tiling_on_sc=False),
    out_type=jax.ShapeDtypeStruct((NW, L), jnp.float32),
    scratch_types=[
        pltpu.VMEM((BPW,), jnp.int32),
        pltpu.VMEM((BPW,), jnp.int32),
        pltpu.VMEM((BPW * NEG,), jnp.int32),
        pltpu.VMEM((CH, HD), jnp.int32),
        pltpu.VMEM((CH, HD), jnp.int32),
        pltpu.VMEM((CH * NEG, HD), jnp.int32),
        pltpu.VMEM((CH, HD), jnp.int32),
        pltpu.VMEM((CH, HD), jnp.int32),
        pltpu.VMEM((CH * NEG, HD), jnp.int32),
        pltpu.VMEM((L,), jnp.float32),
    ] + [pltpu.SemaphoreType.DMA] * 6,
)(_body)


def _pack(t):
    t16 = t.astype(jnp.bfloat16).reshape(VOCAB, HD, 2)
    return lax.bitcast_convert_type(t16, jnp.int32)


@jax.jit
def kernel(pos_u, pos_v, neg_v, u_emb, v_emb):
    pos_u = pos_u.astype(jnp.int32)
    pos_v = pos_v.astype(jnp.int32)
    neg_f = neg_v.reshape(-1).astype(jnp.int32)
    parts = _sgns(pos_u, pos_v, neg_f, _pack(u_emb), _pack(v_emb))
    return jnp.sum(parts) * (1.0 / BATCH)


# speedup vs baseline: 2.6573x; 2.6573x over previous
import functools

import jax
import jax.numpy as jnp
from jax import lax
from jax.experimental import pallas as pl
from jax.experimental.pallas import tpu as pltpu
from jax.experimental.pallas import tpu_sc as plsc

VOCAB = 1000000
DIM = 64
BATCH = 16384
NEG = 5

NC = 2
NS = 16
NW = NC * NS
L = 16

TW = 2 * DIM
BPW = BATCH // NW
CH = 64
NCH = BPW // CH
NGRP = CH // L
UNROLL = 4


def _softplus(x):
    z = jnp.exp(-jnp.abs(x))
    w = z / (z + 2.0)
    w2 = w * w
    p = 1.0 + w2 * (1.0 / 3.0 + w2 * (1.0 / 5.0 + w2 * (1.0 / 7.0 + w2 * (1.0 / 9.0))))
    return jnp.maximum(x, 0.0) + 2.0 * w * p


def _body(pu2_hbm, pv2_hbm, pn2_hbm, pu_hbm, pv_hbm, pn_hbm,
          u_hbm, v_hbm, out_hbm,
          idx_u2, idx_v2, idx_n2, idx_u, idx_v, idx_n,
          ru0, rv0, rn0, ru1, rv1, rn1, loss_v,
          su0, sv0, sn0, su1, sv1, sn1):
    bufs = ((ru0, rv0, rn0), (ru1, rv1, rn1))
    sems = ((su0, sv0, sn0), (su1, sv1, sn1))

    c_id = lax.axis_index("c")
    s_id = lax.axis_index("s")
    wid = s_id * NC + c_id
    base = wid * BPW

    pltpu.sync_copy(pu2_hbm.at[pl.ds(base, BPW)], idx_u2)
    pltpu.sync_copy(pv2_hbm.at[pl.ds(base, BPW)], idx_v2)
    pltpu.sync_copy(pn2_hbm.at[pl.ds(base * NEG, BPW * NEG)], idx_n2)
    pltpu.sync_copy(pu_hbm.at[pl.ds(base, BPW)], idx_u)
    pltpu.sync_copy(pv_hbm.at[pl.ds(base, BPW)], idx_v)
    pltpu.sync_copy(pn_hbm.at[pl.ds(base * NEG, BPW * NEG)], idx_n)

    lane = lax.iota(jnp.int32, L)
    one = jnp.full((L,), 1, jnp.int32)
    hw = jnp.full((L,), DIM, jnp.int32)
    loss = jnp.zeros((L,), jnp.float32)

    def start_fetch(c, s):
        ru, rv, rn = bufs[s]
        semu, semv, semn = sems[s]
        cps = [
            pltpu.async_copy(u_hbm.at[idx_u2.at[pl.ds(c * CH, CH)]], ru, semu),
            pltpu.async_copy(v_hbm.at[idx_v2.at[pl.ds(c * CH, CH)]], rv, semv),
        ]
        for j in range(NEG):
            cps.append(pltpu.async_copy(
                v_hbm.at[idx_n2.at[pl.ds(c * CH * NEG + j * CH, CH)]],
                rn.at[pl.ds(j * CH, CH)], semn))
        return cps

    pend = {0: start_fetch(0, 0)}

    for c in range(NCH):
        s = c % 2
        if c + 1 < NCH:
            pend[c + 1] = start_fetch(c + 1, 1 - s)
        for cp in pend.pop(c):
            cp.wait()
        ru, rv, rn = bufs[s]

        def group(g, loss):
            rb = g * L + lane
            rbn = [rb * NEG + j for j in range(NEG)]
            cu = (idx_u[pl.ds(c * CH + g * L, L)] & one) * hw
            cv = (idx_v[pl.ds(c * CH + g * L, L)] & one) * hw
            cn = [(plsc.load_gather(idx_n, [c * CH * NEG + rbn[j]]) & one) * hw
                  for j in range(NEG)]

            def dstep(t, accs):
                ap, a0, a1, a2, a3, a4 = accs
                for q in range(UNROLL):
                    d = t * UNROLL + q
                    dc = jnp.broadcast_to(d, (L,))
                    uc = plsc.load_gather(ru, [rb, cu + dc])
                    vc = plsc.load_gather(rv, [rb, cv + dc])
                    ap = ap + uc * vc
                    a0 = a0 + plsc.load_gather(rn, [rbn[0], cn[0] + dc]) * uc
                    a1 = a1 + plsc.load_gather(rn, [rbn[1], cn[1] + dc]) * uc
                    a2 = a2 + plsc.load_gather(rn, [rbn[2], cn[2] + dc]) * uc
                    a3 = a3 + plsc.load_gather(rn, [rbn[3], cn[3] + dc]) * uc
                    a4 = a4 + plsc.load_gather(rn, [rbn[4], cn[4] + dc]) * uc
                return ap, a0, a1, a2, a3, a4

            z = jnp.zeros((L,), jnp.float32)
            ap, a0, a1, a2, a3, a4 = lax.fori_loop(
                0, DIM // UNROLL, dstep, (z,) * 6)

            loss = loss + _softplus(-jnp.clip(ap, -10.0, 10.0))
            for t in (a0, a1, a2, a3, a4):
                loss = loss + _softplus(jnp.clip(t, -10.0, 10.0))
            return loss

        loss = lax.fori_loop(0, NGRP, group, loss)

    loss_v[...] = loss
    pltpu.sync_copy(loss_v, out_hbm.at[wid])


_mesh = plsc.VectorSubcoreMesh(core_axis_name="c", subcore_axis_name="s")

_sgns = functools.partial(
    pl.kernel,
    mesh=_mesh,
    compiler_params=pltpu.CompilerParams(
        needs_layout_passes=False, use_tc_tiling_on_sc=False),
    out_type=jax.ShapeDtypeStruct((NW, L), jnp.float32),
    scratch_types=[
        pltpu.VMEM((BPW,), jnp.int32),
        pltpu.VMEM((BPW,), jnp.int32),
        pltpu.VMEM((BPW * NEG,), jnp.int32),
        pltpu.VMEM((BPW,), jnp.int32),
        pltpu.VMEM((BPW,), jnp.int32),
        pltpu.VMEM((BPW * NEG,), jnp.int32),
        pltpu.VMEM((CH, TW), jnp.float32),
        pltpu.VMEM((CH, TW), jnp.float32),
        pltpu.VMEM((CH * NEG, TW), jnp.float32),
        pltpu.VMEM((CH, TW), jnp.float32),
        pltpu.VMEM((CH, TW), jnp.float32),
        pltpu.VMEM((CH * NEG, TW), jnp.float32),
        pltpu.VMEM((L,), jnp.float32),
    ] + [pltpu.SemaphoreType.DMA] * 6,
)(_body)


@jax.jit
def kernel(pos_u, pos_v, neg_v, u_emb, v_emb):
    pos_u = pos_u.astype(jnp.int32)
    pos_v = pos_v.astype(jnp.int32)
    neg_f = neg_v.reshape(-1).astype(jnp.int32)
    u2 = u_emb.reshape(VOCAB // 2, TW)
    v2 = v_emb.reshape(VOCAB // 2, TW)
    parts = _sgns(pos_u >> 1, pos_v >> 1, neg_f >> 1,
                  pos_u, pos_v, neg_f, u2, v2)
    return jnp.sum(parts) * (1.0 / BATCH)


# speedup vs baseline: 3.2314x; 1.2160x over previous
import functools

import jax
import jax.numpy as jnp
from jax import lax
from jax.experimental import pallas as pl
from jax.experimental.pallas import tpu as pltpu
from jax.experimental.pallas import tpu_sc as plsc

VOCAB = 1000000
DIM = 64
BATCH = 16384
NEG = 5

NC = 2
NS = 16
NW = NC * NS
L = 16

BPW = BATCH // NW
CH = 64
NCH = BPW // CH
NGRP = CH // L
UNROLL = 4
W = 2 * DIM


def _softplus(x):
    z = jnp.exp(-jnp.abs(x))
    w = z / (z + 2.0)
    w2 = w * w
    p = 1.0 + w2 * (1.0 / 3.0 + w2 * (1.0 / 5.0 + w2 * (1.0 / 7.0 + w2 * (1.0 / 9.0))))
    return jnp.maximum(x, 0.0) + 2.0 * w * p


def _body(pos_u_hbm, pos_v_hbm, neg_hbm, tbl_hbm, out_hbm,
          idx_u, idx_v, idx_n,
          ru0, rv0, rn0, ru1, rv1, rn1, loss_v,
          su0, sv0, sn0, su1, sv1, sn1):
    bufs = ((ru0, rv0, rn0), (ru1, rv1, rn1))
    sems = ((su0, sv0, sn0), (su1, sv1, sn1))

    c_id = lax.axis_index("c")
    s_id = lax.axis_index("s")
    wid = s_id * NC + c_id
    base = wid * BPW

    pltpu.sync_copy(pos_u_hbm.at[pl.ds(base, BPW)], idx_u)
    pltpu.sync_copy(pos_v_hbm.at[pl.ds(base, BPW)], idx_v)
    pltpu.sync_copy(neg_hbm.at[pl.ds(base * NEG, BPW * NEG)], idx_n)

    lane = lax.iota(jnp.int32, L)
    loss = jnp.zeros((L,), jnp.float32)

    def start_fetch(c, s):
        ru, rv, rn = bufs[s]
        semu, semv, semn = sems[s]
        cps = [
            pltpu.async_copy(tbl_hbm.at[idx_u.at[pl.ds(c * CH, CH)]], ru, semu),
            pltpu.async_copy(tbl_hbm.at[idx_v.at[pl.ds(c * CH, CH)]], rv, semv),
        ]
        for j in range(NEG):
            cps.append(pltpu.async_copy(
                tbl_hbm.at[idx_n.at[pl.ds(c * CH * NEG + j * CH, CH)]],
                rn.at[pl.ds(j * CH, CH)], semn))
        return cps

    pend = {0: start_fetch(0, 0)}

    for c in range(NCH):
        s = c % 2
        if c + 1 < NCH:
            pend[c + 1] = start_fetch(c + 1, 1 - s)
        for cp in pend.pop(c):
            cp.wait()
        ru, rv, rn = bufs[s]

        def group(g, loss):
            rb = g * L + lane
            rbn = [rb * NEG + j for j in range(NEG)]

            def dstep(t, accs):
                ap, a0, a1, a2, a3, a4 = accs
                for q in range(UNROLL):
                    d = t * UNROLL + q
                    dc = jnp.broadcast_to(d, (L,))
                    dv = jnp.broadcast_to(d + DIM, (L,))
                    uc = plsc.load_gather(ru, [rb, dc])
                    vc = plsc.load_gather(rv, [rb, dv])
                    ap = ap + uc * vc
                    a0 = a0 + plsc.load_gather(rn, [rbn[0], dv]) * uc
                    a1 = a1 + plsc.load_gather(rn, [rbn[1], dv]) * uc
                    a2 = a2 + plsc.load_gather(rn, [rbn[2], dv]) * uc
                    a3 = a3 + plsc.load_gather(rn, [rbn[3], dv]) * uc
                    a4 = a4 + plsc.load_gather(rn, [rbn[4], dv]) * uc
                return ap, a0, a1, a2, a3, a4

            z = jnp.zeros((L,), jnp.float32)
            ap, a0, a1, a2, a3, a4 = lax.fori_loop(
                0, DIM // UNROLL, dstep, (z,) * 6)

            loss = loss + _softplus(-jnp.clip(ap, -10.0, 10.0))
            for t in (a0, a1, a2, a3, a4):
                loss = loss + _softplus(jnp.clip(t, -10.0, 10.0))
            return loss

        loss = lax.fori_loop(0, NGRP, group, loss)

    loss_v[...] = loss
    pltpu.sync_copy(loss_v, out_hbm.at[wid])


_mesh = plsc.VectorSubcoreMesh(core_axis_name="c", subcore_axis_name="s")

_sgns = functools.partial(
    pl.kernel,
    mesh=_mesh,
    compiler_params=pltpu.CompilerParams(
        needs_layout_passes=False, use_tc_tiling_on_sc=True),
    out_type=jax.ShapeDtypeStruct((NW, L), jnp.float32),
    scratch_types=[
        pltpu.VMEM((BPW,), jnp.int32),
        pltpu.VMEM((BPW,), jnp.int32),
        pltpu.VMEM((BPW * NEG,), jnp.int32),
        pltpu.VMEM((CH, W), jnp.float32),
        pltpu.VMEM((CH, W), jnp.float32),
        pltpu.VMEM((CH * NEG, W), jnp.float32),
        pltpu.VMEM((CH, W), jnp.float32),
        pltpu.VMEM((CH, W), jnp.float32),
        pltpu.VMEM((CH * NEG, W), jnp.float32),
        pltpu.VMEM((L,), jnp.float32),
    ] + [pltpu.SemaphoreType.DMA] * 6,
)(_body)


@jax.jit
def kernel(pos_u, pos_v, neg_v, u_emb, v_emb):
    pos_u = pos_u.astype(jnp.int32)
    pos_v = pos_v.astype(jnp.int32)
    neg_f = neg_v.reshape(-1).astype(jnp.int32)
    tbl = jnp.concatenate([u_emb, v_emb], axis=1)
    parts = _sgns(pos_u, pos_v, neg_f, tbl)
    return jnp.sum(parts) * (1.0 / BATCH)
